# baseline (speedup 1.0000x reference)
import functools

import jax
import jax.numpy as jnp
from jax import lax
from jax.experimental import pallas as pl
from jax.experimental.pallas import tpu as pltpu
from jax.experimental.pallas import tpu_sc as plsc

_PATCH = 4
_D = 32
_EMB_ROWS = 256
_L = 16


def _make_patch_embed(B, S, nw, nc):
    wpb = nw // B
    rows_w = S // wpb
    groups = rows_w // _L
    win = rows_w + 8
    mesh = plsc.VectorSubcoreMesh(core_axis_name="c", subcore_axis_name="s")

    @functools.partial(
        pl.kernel,
        mesh=mesh,
        compiler_params=pltpu.CompilerParams(use_tc_tiling_on_sc=False, needs_layout_passes=False),
        out_type=jax.ShapeDtypeStruct((nw, rows_w * _D), jnp.float32),
        scratch_types=[
            pltpu.VMEM((win,), jnp.int32),
            pltpu.VMEM((_EMB_ROWS * _D,), jnp.float32),
            pltpu.VMEM((_D,), jnp.float32),
            pltpu.VMEM((rows_w * _D,), jnp.float32),
            pltpu.SemaphoreType.DMA,
            pltpu.SemaphoreType.DMA,
        ],
    )
    def patch_embed(x_hbm, emb_hbm, pos_hbm, pad_hbm, out_hbm,
                    xwin_v, tbl_v, pos_v, rows_v, xsem, tsem):
        wid = lax.axis_index("s") * nc + lax.axis_index("c")
        b = wid // wpb
        r = wid % wpb
        p0 = r * rows_w

        tcp = pltpu.async_copy(emb_hbm, tbl_v, tsem)
        pcp = pltpu.async_copy(pos_hbm, pos_v, tsem)
        start = pl.multiple_of(jnp.maximum(p0 - 8, 0), 8)

        @pl.when(r == 0)
        def _():
            pltpu.async_copy(
                x_hbm.at[b, pl.ds(0, rows_w)], xwin_v.at[pl.ds(0, rows_w)], xsem
            ).wait()

        @pl.when(r != 0)
        def _():
            pltpu.async_copy(x_hbm.at[b, pl.ds(start, win)], xwin_v, xsem).wait()

        tcp.wait()
        pcp.wait()
        pos_lo = pos_v[pl.ds(0, _L)]
        pos_hi = pos_v[pl.ds(_L, _L)]

        def fuse_body(i, _):
            o = i * _D
            tbl_v[pl.ds(o, _L)] = tbl_v[pl.ds(o, _L)] + pos_lo
            tbl_v[pl.ds(o + _L, _L)] = tbl_v[pl.ds(o + _L, _L)] + pos_hi
            return 0

        lax.fori_loop(0, _EMB_ROWS, fuse_body, 0, unroll=4)

        iota = lax.iota(jnp.int32, _L)
        viota_d = iota * _D
        shift = jnp.where(r == 0, -_PATCH, _PATCH)

        def gather_body(g, _):
            offs = jnp.maximum(iota + (g * _L + shift), 0)
            rowidx = plsc.load_gather(xwin_v, [offs])
            tbase = rowidx * _D
            obase = viota_d + g * (_L * _D)
            for c in range(_D):
                v = plsc.load_gather(tbl_v, [tbase + c])
                plsc.store_scatter(rows_v, [obase + c], v)
            return 0

        lax.fori_loop(0, groups, gather_body, 0)

        @pl.when(r == 0)
        def _():
            pltpu.sync_copy(pad_hbm, rows_v.at[pl.ds(0, _PATCH * _D)])

        pltpu.sync_copy(rows_v, out_hbm.at[wid])

    return patch_embed


def kernel(x, emb_table, global_pos_embed, global_pad):
    B, S = x.shape
    assert S % _PATCH == 0
    info = plsc.get_sparse_core_info()
    nc = info.num_cores
    nw = nc * info.num_subcores
    assert nw % B == 0 and S % (nw // B) == 0

    out = _make_patch_embed(B, S, nw, nc)(
        x.astype(jnp.int32),
        emb_table.reshape(-1),
        global_pos_embed.reshape(-1),
        global_pad.reshape(-1),
    )
    return out.reshape(B, S // _PATCH, _PATCH * _D)


# speedup vs baseline: 2.1021x; 2.1021x over previous
import functools

import jax
import jax.numpy as jnp
from jax import lax
from jax.experimental import pallas as pl
from jax.experimental.pallas import tpu as pltpu
from jax.experimental.pallas import tpu_sc as plsc

_PATCH = 4
_D = 32
_EMB_ROWS = 256
_L = 16


def _make_patch_embed(B, S, nw, nc):
    wpb = nw // B
    rows_w = S // wpb
    groups = rows_w // _L
    win = rows_w + 8
    mesh = plsc.VectorSubcoreMesh(core_axis_name="c", subcore_axis_name="s")

    @functools.partial(
        pl.kernel,
        mesh=mesh,
        compiler_params=pltpu.CompilerParams(
            use_tc_tiling_on_sc=False, needs_layout_passes=False
        ),
        out_type=jax.ShapeDtypeStruct((nw, rows_w, _D), jnp.float32),
        scratch_types=[
            pltpu.VMEM((win,), jnp.int32),
            pltpu.VMEM((rows_w,), jnp.int32),
            pltpu.VMEM((_D,), jnp.float32),
            pltpu.VMEM((rows_w, _D), jnp.float32),
            pltpu.SemaphoreType.DMA,
            pltpu.SemaphoreType.DMA,
        ],
    )
    def patch_embed(emb_hbm, x_hbm, pos_hbm, pad_hbm, out_hbm,
                    xwin_v, idx_v, pos_v, rows_v, xsem, gsem):
        wid = lax.axis_index("s") * nc + lax.axis_index("c")
        b = wid // wpb
        r = wid % wpb
        p0 = r * rows_w
        start = pl.multiple_of(jnp.maximum(p0 - 8, 0), 8)

        pcp = pltpu.async_copy(pos_hbm, pos_v, gsem)

        @pl.when(r == 0)
        def _():
            pltpu.async_copy(
                x_hbm.at[b, pl.ds(0, rows_w)], xwin_v.at[pl.ds(0, rows_w)], xsem
            ).wait()

        @pl.when(r != 0)
        def _():
            pltpu.async_copy(x_hbm.at[b, pl.ds(start, win)], xwin_v, xsem).wait()

        iota = lax.iota(jnp.int32, _L)
        shift = jnp.where(r == 0, -_PATCH, _PATCH)

        def idx_body(g, _):
            offs = jnp.maximum(iota + (g * _L + shift), 0)
            idx_v[pl.ds(g * _L, _L)] = plsc.load_gather(xwin_v, [offs])
            return 0

        lax.fori_loop(0, groups, idx_body, 0)

        pcp.wait()
        pos_lo = pos_v[pl.ds(0, _L)]
        pos_hi = pos_v[pl.ds(_L, _L)]

        def fill_body(i, _):
            rows_v[i, pl.ds(0, _L)] = pos_lo
            rows_v[i, pl.ds(_L, _L)] = pos_hi
            return 0

        lax.fori_loop(0, rows_w, fill_body, 0, unroll=8)

        pltpu.async_copy(emb_hbm.at[idx_v], rows_v, gsem, add=True).wait()

        @pl.when(r == 0)
        def _():
            pltpu.sync_copy(pad_hbm, rows_v.at[pl.ds(0, _PATCH)])

        pltpu.sync_copy(rows_v, out_hbm.at[wid])

    return patch_embed


def kernel(x, emb_table, global_pos_embed, global_pad):
    B, S = x.shape
    assert S % _PATCH == 0
    info = plsc.get_sparse_core_info()
    nc = info.num_cores
    nw = nc * info.num_subcores
    assert nw % B == 0 and S % (nw // B) == 0

    out = _make_patch_embed(B, S, nw, nc)(
        emb_table,
        x.astype(jnp.int32),
        global_pos_embed.reshape(_D),
        global_pad.reshape(_PATCH, _D),
    )
    return out.reshape(B, S // _PATCH, _PATCH * _D)
